# baseline (speedup 1.0000x reference)
import functools

import jax
import jax.numpy as jnp
from jax import lax
from jax.experimental import pallas as pl
from jax.experimental.pallas import tpu as pltpu
from jax.experimental.pallas import tpu_sc as plsc

N = 10000
NPAD = 10240
D = 128
E = 320000
NC = 2
NS = 16
NW = NC * NS
EPW = E // NW
KB = 80
NBLK = EPW // KB
NSLOT = 3
RPT = NPAD // NS
XR0 = (NS - 1) * (NPAD // NS)
CHUNK = 128
NCHUNK = RPT // CHUNK
LANES = 16
RB = 1000
GRID = N // RB


def _post_body(p_ref, degp_ref, x_ref, w1_ref, w2_ref, b_ref, o_ref):
    s = p_ref[0].astype(jnp.float32) + p_ref[1].astype(jnp.float32)
    d = degp_ref[0] + degp_ref[1]
    dcol = d[:, 0:1]
    xb = x_ref[...]
    dn = (((1,), (1,)), ((), ()))
    msg1 = lax.dot_general(s, w1_ref[...], dn,
                           preferred_element_type=jnp.float32)
    bp = (
        lax.dot_general(xb, w2_ref[...], dn,
                        preferred_element_type=jnp.float32)
        + b_ref[...]
    )
    o_ref[...] = msg1 + xb + dcol * bp


_post_call = pl.pallas_call(
    _post_body,
    grid=(GRID,),
    in_specs=[
        pl.BlockSpec((NC, RB, D), lambda i: (0, i, 0)),
        pl.BlockSpec((NC, RB, LANES), lambda i: (0, i, 0)),
        pl.BlockSpec((RB, D), lambda i: (i, 0)),
        pl.BlockSpec((D, D), lambda i: (0, 0)),
        pl.BlockSpec((D, D), lambda i: (0, 1)),
        pl.BlockSpec((1, D), lambda i: (0, 0)),
    ],
    out_specs=pl.BlockSpec((RB, D), lambda i: (i, 0)),
    out_shape=jax.ShapeDtypeStruct((N, D), jnp.float32),
)


_MESH = plsc.VectorSubcoreMesh(
    core_axis_name="c", subcore_axis_name="s", num_cores=NC, num_subcores=NS
)


def _fill_rows(ref, nrows, ncols, val, dtype):
    lanes = 32 if dtype == jnp.bfloat16 else LANES
    vec = jnp.full((lanes,), val, dtype)

    def body(i, carry):
        for j in range(ncols // lanes):
            ref[i, pl.ds(j * lanes, lanes)] = vec
        return carry

    lax.fori_loop(0, nrows, body, 0)


@functools.partial(
    pl.kernel,
    out_type=[
        jax.ShapeDtypeStruct((NC, NPAD, D), jnp.bfloat16),
        jax.ShapeDtypeStruct((NC, NPAD, LANES), jnp.float32),
    ],
    mesh=_MESH,
    compiler_params=pltpu.CompilerParams(use_tc_tiling_on_sc=False),
    scratch_types=[
        pltpu.VMEM((NBLK, KB), jnp.int32),
        pltpu.VMEM((NBLK, KB), jnp.int32),
        [pltpu.VMEM((KB, D), jnp.bfloat16)] * NSLOT,
        pltpu.VMEM((KB, LANES), jnp.float32),
        pltpu.VMEM_SHARED((NPAD, D), jnp.bfloat16),
        pltpu.VMEM_SHARED((NPAD, D), jnp.bfloat16),
        pltpu.VMEM_SHARED((NPAD, LANES), jnp.float32),
        [pltpu.SemaphoreType.DMA] * NSLOT,
        [pltpu.SemaphoreType.DMA] * NSLOT,
        [pltpu.SemaphoreType.DMA] * NSLOT,
    ],
)
def _sc_scatter(x_hbm, row_hbm, col_hbm, out_hbm, deg_hbm,
                row_v, col_v, gs, ones_v, xtab, acc, dacc,
                gsems, ssems, osems):
    cid = lax.axis_index("c")
    sid = lax.axis_index("s")
    wid = sid * NC + cid
    base = sid * RPT

    pltpu.sync_copy(row_hbm.at[wid], row_v)
    pltpu.sync_copy(col_hbm.at[wid], col_v)

    @pl.when(sid < NS - 1)
    def _():
        pltpu.sync_copy(x_hbm.at[pl.ds(base, RPT)], xtab.at[pl.ds(base, RPT)])

    @pl.when(sid == NS - 1)
    def _():
        pltpu.sync_copy(x_hbm.at[pl.ds(XR0, N - XR0)],
                        xtab.at[pl.ds(XR0, N - XR0)])

    _fill_rows(gs[0], KB, D, 0.0, jnp.bfloat16)
    _fill_rows(ones_v, KB, LANES, 0.0, jnp.float32)
    for ch in range(RPT // KB):
        cb = base + ch * KB
        pltpu.sync_copy(gs[0], acc.at[pl.ds(cb, KB)])
        pltpu.sync_copy(ones_v, dacc.at[pl.ds(cb, KB), :])
    _fill_rows(ones_v, KB, LANES, 1.0, jnp.float32)
    plsc.subcore_barrier()

    def wait_gather(b):
        pltpu.make_async_copy(x_hbm.at[pl.ds(0, KB)], gs[b], gsems[b]).wait()

    def wait_scatter(b):
        pltpu.make_async_copy(x_hbm.at[pl.ds(0, KB)], gs[b], ssems[b]).wait()

    def wait_ones(b):
        pltpu.make_async_copy(
            deg_hbm.at[cid, pl.ds(0, KB)], ones_v, osems[b]
        ).wait()

    def start_gather(b, j):
        pltpu.async_copy(xtab.at[row_v.at[j]], gs[b], gsems[b])

    def visit(b, j):
        wait_gather(b)
        pltpu.async_copy(gs[b], acc.at[col_v.at[j]], ssems[b], add=True)
        pltpu.async_copy(ones_v, dacc.at[col_v.at[j]], osems[b], add=True)

    for b in range(NSLOT):
        start_gather(b, b)

    def body(i, carry):
        for b in range(NSLOT):
            visit(b, NSLOT * i + b)
        for b in range(NSLOT):
            j = NSLOT * i + b
            wait_scatter(b)
            wait_ones(b)

            @pl.when(j + NSLOT < NBLK)
            def _():
                start_gather(b, j + NSLOT)
        return carry

    lax.fori_loop(0, NBLK // NSLOT, body, 0)
    for t in range(NBLK - (NBLK // NSLOT) * NSLOT):
        visit(t, (NBLK // NSLOT) * NSLOT + t)
        wait_scatter(t)
        wait_ones(t)
    plsc.subcore_barrier()

    pltpu.sync_copy(acc.at[pl.ds(base, RPT)], out_hbm.at[cid, pl.ds(base, RPT)])
    pltpu.sync_copy(dacc.at[pl.ds(base, RPT)], deg_hbm.at[cid, pl.ds(base, RPT)])


def kernel(x, edge_index, W_msg, b_msg):
    xb = x.astype(jnp.bfloat16)
    row3 = edge_index[0].reshape(NW, NBLK, KB)
    col3 = edge_index[1].reshape(NW, NBLK, KB)
    partials, degp = _sc_scatter(xb, row3, col3)
    return _post_call(partials, degp, x, W_msg, W_msg, b_msg.reshape(1, D))


# speedup vs baseline: 1.1169x; 1.1169x over previous
import functools

import jax
import jax.numpy as jnp
from jax import lax
from jax.experimental import pallas as pl
from jax.experimental.pallas import tpu as pltpu
from jax.experimental.pallas import tpu_sc as plsc

N = 10000
NPAD = 10240
D = 128
E = 320000
NC = 2
NS = 16
NW = NC * NS
EPW = E // NW
KB = 80
NBLK = EPW // KB
NSLOT = 3
RPT = NPAD // NS
XR0 = (NS - 1) * (NPAD // NS)
CHUNK = 128
NCHUNK = RPT // CHUNK
LANES = 16
RB = 1000
GRID = N // RB


def _post_body(p_ref, degp_ref, x_ref, w1_ref, w2_ref, b_ref, o_ref):
    s = p_ref[0].astype(jnp.float32) + p_ref[1].astype(jnp.float32)
    d = degp_ref[0] + degp_ref[1]
    dcol = d[:, 0:1]
    xb = x_ref[...]
    dn = (((1,), (1,)), ((), ()))
    msg1 = lax.dot_general(s, w1_ref[...], dn,
                           preferred_element_type=jnp.float32)
    bp = (
        lax.dot_general(xb, w2_ref[...], dn,
                        preferred_element_type=jnp.float32)
        + b_ref[...]
    )
    o_ref[...] = msg1 + xb + dcol * bp


_post_call = pl.pallas_call(
    _post_body,
    grid=(GRID,),
    in_specs=[
        pl.BlockSpec((NC, RB, D), lambda i: (0, i, 0)),
        pl.BlockSpec((NC, RB, LANES), lambda i: (0, i, 0)),
        pl.BlockSpec((RB, D), lambda i: (i, 0)),
        pl.BlockSpec((D, D), lambda i: (0, 0)),
        pl.BlockSpec((D, D), lambda i: (0, 1)),
        pl.BlockSpec((1, D), lambda i: (0, 0)),
    ],
    out_specs=pl.BlockSpec((RB, D), lambda i: (i, 0)),
    out_shape=jax.ShapeDtypeStruct((N, D), jnp.float32),
)


_MESH = plsc.VectorSubcoreMesh(
    core_axis_name="c", subcore_axis_name="s", num_cores=NC, num_subcores=NS
)


def _fill_rows(ref, nrows, ncols, val, dtype):
    lanes = 32 if dtype == jnp.bfloat16 else LANES
    vec = jnp.full((lanes,), val, dtype)

    def body(i, carry):
        for j in range(ncols // lanes):
            ref[i, pl.ds(j * lanes, lanes)] = vec
        return carry

    lax.fori_loop(0, nrows, body, 0)


@functools.partial(
    pl.kernel,
    out_type=[
        jax.ShapeDtypeStruct((NC, NPAD, D), jnp.bfloat16),
        jax.ShapeDtypeStruct((NC, NPAD, LANES), jnp.float32),
    ],
    mesh=_MESH,
    compiler_params=pltpu.CompilerParams(use_tc_tiling_on_sc=False),
    scratch_types=[
        pltpu.VMEM((NBLK, KB), jnp.int32),
        pltpu.VMEM((NBLK, KB), jnp.int32),
        [pltpu.VMEM((KB, D), jnp.bfloat16)] * NSLOT,
        pltpu.VMEM((KB, LANES), jnp.float32),
        pltpu.VMEM_SHARED((NPAD, D), jnp.bfloat16),
        pltpu.VMEM_SHARED((NPAD, D), jnp.bfloat16),
        pltpu.VMEM_SHARED((NPAD, LANES), jnp.float32),
        [pltpu.SemaphoreType.DMA] * NSLOT,
        [pltpu.SemaphoreType.DMA] * NSLOT,
        [pltpu.SemaphoreType.DMA] * NSLOT,
    ],
)
def _sc_scatter(x_hbm, row_hbm, col_hbm, out_hbm, deg_hbm,
                row_v, col_v, gs, ones_v, xtab, acc, dacc,
                gsems, ssems, osems):
    cid = lax.axis_index("c")
    sid = lax.axis_index("s")
    wid = sid * NC + cid
    base = sid * RPT

    pltpu.sync_copy(row_hbm.at[wid], row_v)
    pltpu.sync_copy(col_hbm.at[wid], col_v)

    @pl.when(sid < NS - 1)
    def _():
        pltpu.sync_copy(x_hbm.at[pl.ds(base, RPT)], xtab.at[pl.ds(base, RPT)])

    @pl.when(sid == NS - 1)
    def _():
        pltpu.sync_copy(x_hbm.at[pl.ds(XR0, N - XR0)],
                        xtab.at[pl.ds(XR0, N - XR0)])

    _fill_rows(gs[0], KB, D, 0.0, jnp.bfloat16)
    _fill_rows(ones_v, KB, LANES, 0.0, jnp.float32)
    for ch in range(RPT // KB):
        cb = base + ch * KB
        pltpu.sync_copy(gs[0], acc.at[pl.ds(cb, KB)])
        pltpu.sync_copy(ones_v, dacc.at[pl.ds(cb, KB), :])
    _fill_rows(ones_v, KB, LANES, 1.0, jnp.float32)
    plsc.subcore_barrier()

    def wait_gather(b):
        pltpu.make_async_copy(x_hbm.at[pl.ds(0, KB)], gs[b], gsems[b]).wait()

    def wait_scatter(b):
        pltpu.make_async_copy(x_hbm.at[pl.ds(0, KB)], gs[b], ssems[b]).wait()

    def wait_ones(b):
        pltpu.make_async_copy(
            deg_hbm.at[cid, pl.ds(0, KB)], ones_v, osems[b]
        ).wait()

    def start_gather(b, j):
        pltpu.async_copy(xtab.at[row_v.at[j]], gs[b], gsems[b])

    def visit(b, j):
        wait_gather(b)

        @pl.when(j >= NSLOT)
        def _():
            wait_ones(b)

        pltpu.async_copy(gs[b], acc.at[col_v.at[j]], ssems[b], add=True)
        pltpu.async_copy(ones_v, dacc.at[col_v.at[j]], osems[b], add=True)

    for b in range(NSLOT):
        start_gather(b, b)

    def body(i, carry):
        for b in range(NSLOT):
            j = NSLOT * i + b
            visit(b, j)
            wait_scatter(b)

            @pl.when(j + NSLOT < NBLK)
            def _():
                start_gather(b, j + NSLOT)
        return carry

    lax.fori_loop(0, NBLK // NSLOT, body, 0)
    for t in range(NBLK - (NBLK // NSLOT) * NSLOT):
        visit(t, (NBLK // NSLOT) * NSLOT + t)
        wait_scatter(t)

    for b in range(NSLOT):
        wait_ones(b)
    plsc.subcore_barrier()

    pltpu.sync_copy(acc.at[pl.ds(base, RPT)], out_hbm.at[cid, pl.ds(base, RPT)])
    pltpu.sync_copy(dacc.at[pl.ds(base, RPT)], deg_hbm.at[cid, pl.ds(base, RPT)])


def kernel(x, edge_index, W_msg, b_msg):
    xb = x.astype(jnp.bfloat16)
    row3 = edge_index[0].reshape(NW, NBLK, KB)
    col3 = edge_index[1].reshape(NW, NBLK, KB)
    partials, degp = _sc_scatter(xb, row3, col3)
    return _post_call(partials, degp, x, W_msg, W_msg, b_msg.reshape(1, D))
